# baseline (speedup 1.0000x reference)
import functools

import jax
import jax.numpy as jnp
from jax import lax
from jax.experimental import pallas as pl
from jax.experimental.pallas import tpu as pltpu
from jax.experimental.pallas import tpu_sc as plsc

_NR = 5000
_N = 10000
_E = 320000
_D = 128
_NT = 32
_EPT = _E // _NT
_NPAD = 10240
_S = 2 * _NPAD
_EB = _S // _NT
_EB0 = 768
_EB1 = 512
_EBM = max(_EB0, _EB1)
_GCH = 128
_RPAD = 10080
_BLK = 1000
_RBLK = _RPAD // 10

_mesh = plsc.VectorSubcoreMesh(core_axis_name="c", subcore_axis_name="s")
_sc_params = pltpu.CompilerParams(needs_layout_passes=False)


@functools.partial(
    pl.kernel,
    out_type=[
        jax.ShapeDtypeStruct((_NT, _S), jnp.int32),
        jax.ShapeDtypeStruct((_NT, _S), jnp.int32),
    ],
    mesh=_mesh,
    compiler_params=_sc_params,
    scratch_types=[
        pltpu.VMEM((_EPT,), jnp.int32),
        pltpu.VMEM((_EPT,), jnp.int32),
        pltpu.VMEM((_EPT,), jnp.int32),
        pltpu.VMEM((_S,), jnp.int32),
        pltpu.VMEM((_S,), jnp.int32),
    ],
)
def _phase_a(src_h, rel_h, des_h, pk_h, cnt_h,
             src_v, rel_v, des_v, pk_v, cnt_v):
    wid = lax.axis_index("s") * 2 + lax.axis_index("c")
    base = wid * _EPT
    pltpu.sync_copy(src_h.at[pl.ds(base, _EPT)], src_v)
    pltpu.sync_copy(rel_h.at[pl.ds(base, _EPT)], rel_v)
    pltpu.sync_copy(des_h.at[pl.ds(base, _EPT)], des_v)

    zero = jnp.zeros((16,), jnp.int32)

    def init_body(i, _):
        for k in range(8):
            cnt_v[pl.ds((i * 8 + k) * 16, 16)] = zero
        return 0

    lax.fori_loop(0, _S // 128, init_body, 0)

    def body(i, _):
        for k in range(5):
            off = (i * 5 + k) * 16
            sl = pl.ds(off, 16)
            s = src_v[sl]
            r = rel_v[sl]
            d = des_v[sl]
            key = jnp.where(r < _NR, d, d + _NPAD)
            c16, last = plsc.scan_count(key)
            p = (s << 14) | r
            plsc.store_scatter(pk_v, [key], p, mask=last)
            plsc.addupdate_scatter(cnt_v, [key], c16, mask=last)
        return 0

    lax.fori_loop(0, _EPT // 80, body, 0)

    pltpu.sync_copy(pk_v, pk_h.at[wid])
    pltpu.sync_copy(cnt_v, cnt_h.at[wid])


@functools.partial(
    pl.kernel,
    out_type=[
        jax.ShapeDtypeStruct((2, _NPAD, _D), jnp.float32),
        jax.ShapeDtypeStruct((2, _NPAD), jnp.float32),
    ],
    mesh=_mesh,
    compiler_params=_sc_params,
    scratch_types=[
        pltpu.VMEM((_NT, _EBM), jnp.int32),
        pltpu.VMEM((_NT, _EBM), jnp.int32),
        pltpu.VMEM((_EBM,), jnp.int32),
        pltpu.VMEM((_EBM,), jnp.int32),
        pltpu.VMEM((_EBM,), jnp.float32),
        pltpu.VMEM((_GCH, _D), jnp.float32),
        pltpu.VMEM((_GCH, _D), jnp.float32),
        pltpu.SemaphoreType.DMA,
        pltpu.SemaphoreType.DMA,
    ],
)
def _phase_b(pk_h, cnt_h, node_h, rela_h, u_h, c_h,
             pks, cnts, bsrc, brel, cf, rows_n, rows_r,
             sem1, sem2):
    c = lax.axis_index("c")
    s = lax.axis_index("s")
    col0 = jnp.where(c == 0, s * _EB0, 16 * _EB0 + s * _EB1)

    @pl.when(c == 0)
    def _():
        pltpu.sync_copy(pk_h.at[:, pl.ds(col0, _EB0)], pks)
        pltpu.sync_copy(cnt_h.at[:, pl.ds(col0, _EB0)], cnts)

    @pl.when(c == 1)
    def _():
        pltpu.sync_copy(pk_h.at[:, pl.ds(col0, _EB1)],
                        pks.at[:, pl.ds(0, _EB1)])
        pltpu.sync_copy(cnt_h.at[:, pl.ds(col0, _EB1)],
                        cnts.at[:, pl.ds(0, _EB1)])

    def mbody(i, _):
        off = i * 16
        z = jnp.zeros((16,), jnp.int32)
        bp = z
        cs = z
        for t in range(_NT):
            cv = cnts[t, pl.ds(off, 16)]
            bp = jnp.where(cv > 0, pks[t, pl.ds(off, 16)], bp)
            cs = cs + cv
        bsrc[pl.ds(off, 16)] = lax.shift_right_logical(bp, 14)
        brel[pl.ds(off, 16)] = bp & 16383
        cf[pl.ds(off, 16)] = cs.astype(jnp.float32)
        return 0

    nvr = jnp.where(c == 0, _EB0 // 16, _EB1 // 16)
    lax.fori_loop(0, nvr, mbody, 0)

    def gbody(g, _):
        goff = g * _GCH
        ent0 = col0 + goff
        half = ent0 // _NPAD
        row = ent0 - half * _NPAD
        cp1 = pltpu.async_copy(
            node_h.at[bsrc.at[pl.ds(goff, _GCH)]], rows_n, sem1)
        cp2 = pltpu.async_copy(
            rela_h.at[brel.at[pl.ds(goff, _GCH)]], rows_r, sem2)
        cp1.wait()
        cp2.wait()

        def abody(j, _):
            for k in range(_D // 16):
                sl = pl.ds(k * 16, 16)
                rows_n[j, sl] = rows_n[j, sl] + rows_r[j, sl]
            return 0

        lax.fori_loop(0, _GCH, abody, 0)
        pltpu.sync_copy(rows_n, u_h.at[half, pl.ds(row, _GCH)])
        pltpu.sync_copy(cf.at[pl.ds(goff, _GCH)],
                        c_h.at[half, pl.ds(row, _GCH)])
        return 0

    nch = jnp.where(c == 0, _EB0 // _GCH, _EB1 // _GCH)
    lax.fori_loop(0, nch, gbody, 0)


_DN = (((1,), (1,)), ((), ()))


def _tc1_body(node, r0, rela, wi, wr, base_o, hr_o):
    nb = node[...] + r0[...]
    base_o[...] = lax.dot_general(
        nb, wi[...], _DN, preferred_element_type=jnp.float32)
    hr_o[...] = lax.dot_general(
        rela[...], wr[...], _DN, preferred_element_type=jnp.float32)


def _tc2_body(base, u1, u2, c1, c2, wo, ws, hv_o):
    acc = base[...]
    acc += c1[...] * lax.dot_general(
        u1[0], wo[...], _DN, preferred_element_type=jnp.float32)
    acc += c2[...] * lax.dot_general(
        u2[0], ws[...], _DN, preferred_element_type=jnp.float32)
    hv_o[...] = acc


_NRE = 2 * _NR + 1

_tc1_call = pl.pallas_call(
    _tc1_body,
    grid=(_N // _BLK,),
    in_specs=[
        pl.BlockSpec((_BLK, _D), lambda i: (i, 0)),
        pl.BlockSpec((1, _D), lambda i: (0, 0)),
        pl.BlockSpec((_RBLK, _D), lambda i: (i, 0)),
        pl.BlockSpec((_D, _D), lambda i: (0, 0)),
        pl.BlockSpec((_D, _D), lambda i: (0, 0)),
    ],
    out_specs=[
        pl.BlockSpec((_BLK, _D), lambda i: (i, 0)),
        pl.BlockSpec((_RBLK, _D), lambda i: (i, 0)),
    ],
    out_shape=[
        jax.ShapeDtypeStruct((_N, _D), jnp.float32),
        jax.ShapeDtypeStruct((_NRE, _D), jnp.float32),
    ],
)

_tc2_call = pl.pallas_call(
    _tc2_body,
    grid=(_N // _BLK,),
    in_specs=[
        pl.BlockSpec((_BLK, _D), lambda i: (i, 0)),
        pl.BlockSpec((1, _BLK, _D), lambda i: (0, i, 0)),
        pl.BlockSpec((1, _BLK, _D), lambda i: (1, i, 0)),
        pl.BlockSpec((_BLK, 1), lambda i: (i, 0)),
        pl.BlockSpec((_BLK, 1), lambda i: (i, 0)),
        pl.BlockSpec((_D, _D), lambda i: (0, 0)),
        pl.BlockSpec((_D, _D), lambda i: (0, 0)),
    ],
    out_specs=pl.BlockSpec((_BLK, _D), lambda i: (i, 0)),
    out_shape=jax.ShapeDtypeStruct((_N, _D), jnp.float32),
)


def kernel(node_embed, rela_embed, edges, W_o, W_i, W_s, W_r):
    pk, cnt = _phase_a(edges[:, 0], edges[:, 1], edges[:, 2])
    u, c = _phase_b(pk, cnt, node_embed, rela_embed)

    r0 = rela_embed[_NRE - 1][None, :]
    base, hr = _tc1_call(node_embed, r0, rela_embed, W_i, W_r)
    c1 = c[0, :_N, None]
    c2 = c[1, :_N, None]
    hv = _tc2_call(base, u, u, c1, c2, W_o, W_s)
    return hv, hr


# speedup vs baseline: 1.0575x; 1.0575x over previous
import functools

import jax
import jax.numpy as jnp
from jax import lax
from jax.experimental import pallas as pl
from jax.experimental.pallas import tpu as pltpu
from jax.experimental.pallas import tpu_sc as plsc

_NR = 5000
_N = 10000
_E = 320000
_D = 128
_NT = 32
_EPT = _E // _NT
_NPAD = 10240
_S = 2 * _NPAD
_EB = _S // _NT
_EB0 = 640
_EB1 = 640
_EBM = max(_EB0, _EB1)
_GCH = 128
_RPAD = 10080
_BLK = 1000
_RBLK = _RPAD // 10

_mesh = plsc.VectorSubcoreMesh(core_axis_name="c", subcore_axis_name="s")
_sc_params = pltpu.CompilerParams(needs_layout_passes=False)


@functools.partial(
    pl.kernel,
    out_type=[
        jax.ShapeDtypeStruct((_NT, _S), jnp.int32),
        jax.ShapeDtypeStruct((_NT, _S), jnp.int32),
    ],
    mesh=_mesh,
    compiler_params=_sc_params,
    scratch_types=[
        pltpu.VMEM((_EPT,), jnp.int32),
        pltpu.VMEM((_EPT,), jnp.int32),
        pltpu.VMEM((_EPT,), jnp.int32),
        pltpu.VMEM((_S,), jnp.int32),
        pltpu.VMEM((_S,), jnp.int32),
    ],
)
def _phase_a(src_h, rel_h, des_h, pk_h, cnt_h,
             src_v, rel_v, des_v, pk_v, cnt_v):
    wid = lax.axis_index("s") * 2 + lax.axis_index("c")
    base = wid * _EPT
    pltpu.sync_copy(src_h.at[pl.ds(base, _EPT)], src_v)
    pltpu.sync_copy(rel_h.at[pl.ds(base, _EPT)], rel_v)
    pltpu.sync_copy(des_h.at[pl.ds(base, _EPT)], des_v)

    zero = jnp.zeros((16,), jnp.int32)

    def init_body(i, _):
        for k in range(8):
            cnt_v[pl.ds((i * 8 + k) * 16, 16)] = zero
        return 0

    lax.fori_loop(0, _S // 128, init_body, 0)

    def body(i, _):
        for k in range(5):
            off = (i * 5 + k) * 16
            sl = pl.ds(off, 16)
            s = src_v[sl]
            r = rel_v[sl]
            d = des_v[sl]
            key = jnp.where(r < _NR, d, d + _NPAD)
            c16, last = plsc.scan_count(key)
            p = (s << 14) | r
            plsc.store_scatter(pk_v, [key], p, mask=last)
            plsc.addupdate_scatter(cnt_v, [key], c16, mask=last)
        return 0

    lax.fori_loop(0, _EPT // 80, body, 0)

    pltpu.sync_copy(pk_v, pk_h.at[wid])
    pltpu.sync_copy(cnt_v, cnt_h.at[wid])


@functools.partial(
    pl.kernel,
    out_type=[
        jax.ShapeDtypeStruct((2, _NPAD, _D), jnp.float32),
        jax.ShapeDtypeStruct((2, _NPAD), jnp.float32),
    ],
    mesh=_mesh,
    compiler_params=_sc_params,
    scratch_types=[
        pltpu.VMEM((_NT, _EB), jnp.int32),
        pltpu.VMEM((_NT, _EB), jnp.int32),
        pltpu.VMEM((_EB,), jnp.int32),
        pltpu.VMEM((_EB,), jnp.int32),
        pltpu.VMEM((_EB,), jnp.float32),
        pltpu.VMEM((_GCH, _D), jnp.float32),
        pltpu.VMEM((_GCH, _D), jnp.float32),
        pltpu.SemaphoreType.DMA,
        pltpu.SemaphoreType.DMA,
    ],
)
def _phase_b(pk_h, cnt_h, node_h, rela_h, u_h, c_h,
             pks, cnts, bsrc, brel, cf, rows_n, rows_r,
             sem1, sem2):
    wid = lax.axis_index("s") * 2 + lax.axis_index("c")
    col0 = wid * _EB
    half = wid // (_NPAD // _EB)
    row0 = col0 - half * _NPAD

    pltpu.sync_copy(pk_h.at[:, pl.ds(col0, _EB)], pks)
    pltpu.sync_copy(cnt_h.at[:, pl.ds(col0, _EB)], cnts)

    def mbody(i, _):
        off = i * 16
        z = jnp.zeros((16,), jnp.int32)
        bp = z
        cs = z
        for t in range(_NT):
            cv = cnts[t, pl.ds(off, 16)]
            bp = jnp.where(cv > 0, pks[t, pl.ds(off, 16)], bp)
            cs = cs + cv
        bsrc[pl.ds(off, 16)] = lax.shift_right_logical(bp, 14)
        brel[pl.ds(off, 16)] = bp & 16383
        cf[pl.ds(off, 16)] = cs.astype(jnp.float32)
        return 0

    lax.fori_loop(0, _EB // 16, mbody, 0)

    pltpu.sync_copy(cf, c_h.at[half, pl.ds(row0, _EB)])

    def gbody(g, _):
        goff = g * _GCH
        cp1 = pltpu.async_copy(
            node_h.at[bsrc.at[pl.ds(goff, _GCH)]], rows_n, sem1)
        cp2 = pltpu.async_copy(
            rela_h.at[brel.at[pl.ds(goff, _GCH)]], rows_r, sem2)
        cp1.wait()
        cp2.wait()

        def abody(j, _):
            for k in range(_D // 16):
                sl = pl.ds(k * 16, 16)
                rows_n[j, sl] = rows_n[j, sl] + rows_r[j, sl]
            return 0

        lax.fori_loop(0, _GCH, abody, 0)
        pltpu.sync_copy(rows_n, u_h.at[half, pl.ds(row0 + goff, _GCH)])
        return 0

    lax.fori_loop(0, _EB // _GCH, gbody, 0)


_DN = (((1,), (1,)), ((), ()))


def _tc1_body(node, r0, rela, wi, wr, base_o, hr_o):
    nb = node[...] + r0[...]
    base_o[...] = lax.dot_general(
        nb, wi[...], _DN, preferred_element_type=jnp.float32)
    hr_o[...] = lax.dot_general(
        rela[...], wr[...], _DN, preferred_element_type=jnp.float32)


def _tc2_body(base, u1, u2, c1, c2, wo, ws, hv_o):
    acc = base[...]
    acc += c1[...] * lax.dot_general(
        u1[0], wo[...], _DN, preferred_element_type=jnp.float32)
    acc += c2[...] * lax.dot_general(
        u2[0], ws[...], _DN, preferred_element_type=jnp.float32)
    hv_o[...] = acc


_NRE = 2 * _NR + 1

_tc1_call = pl.pallas_call(
    _tc1_body,
    grid=(_N // _BLK,),
    in_specs=[
        pl.BlockSpec((_BLK, _D), lambda i: (i, 0)),
        pl.BlockSpec((1, _D), lambda i: (0, 0)),
        pl.BlockSpec((_RBLK, _D), lambda i: (i, 0)),
        pl.BlockSpec((_D, _D), lambda i: (0, 0)),
        pl.BlockSpec((_D, _D), lambda i: (0, 0)),
    ],
    out_specs=[
        pl.BlockSpec((_BLK, _D), lambda i: (i, 0)),
        pl.BlockSpec((_RBLK, _D), lambda i: (i, 0)),
    ],
    out_shape=[
        jax.ShapeDtypeStruct((_N, _D), jnp.float32),
        jax.ShapeDtypeStruct((_NRE, _D), jnp.float32),
    ],
)

_tc2_call = pl.pallas_call(
    _tc2_body,
    grid=(_N // _BLK,),
    in_specs=[
        pl.BlockSpec((_BLK, _D), lambda i: (i, 0)),
        pl.BlockSpec((1, _BLK, _D), lambda i: (0, i, 0)),
        pl.BlockSpec((1, _BLK, _D), lambda i: (1, i, 0)),
        pl.BlockSpec((_BLK, 1), lambda i: (i, 0)),
        pl.BlockSpec((_BLK, 1), lambda i: (i, 0)),
        pl.BlockSpec((_D, _D), lambda i: (0, 0)),
        pl.BlockSpec((_D, _D), lambda i: (0, 0)),
    ],
    out_specs=pl.BlockSpec((_BLK, _D), lambda i: (i, 0)),
    out_shape=jax.ShapeDtypeStruct((_N, _D), jnp.float32),
)


def kernel(node_embed, rela_embed, edges, W_o, W_i, W_s, W_r):
    pk, cnt = _phase_a(edges[:, 0], edges[:, 1], edges[:, 2])
    u, c = _phase_b(pk, cnt, node_embed, rela_embed)

    r0 = rela_embed[_NRE - 1][None, :]
    base, hr = _tc1_call(node_embed, r0, rela_embed, W_i, W_r)
    c1 = c[0, :_N, None]
    c2 = c[1, :_N, None]
    hv = _tc2_call(base, u, u, c1, c2, W_o, W_s)
    return hv, hr


# speedup vs baseline: 1.0616x; 1.0039x over previous
import functools

import jax
import jax.numpy as jnp
from jax import lax
from jax.experimental import pallas as pl
from jax.experimental.pallas import tpu as pltpu
from jax.experimental.pallas import tpu_sc as plsc

_NR = 5000
_N = 10000
_E = 320000
_D = 128
_NT = 32
_EPT = _E // _NT
_NPAD = 10240
_S = 2 * _NPAD
_EB = _S // _NT
_GCH = 128
_RPAD = 10080
_BLK = 1000
_RBLK = _RPAD // 10

_mesh = plsc.VectorSubcoreMesh(core_axis_name="c", subcore_axis_name="s")
_sc_params = pltpu.CompilerParams(needs_layout_passes=False)


@functools.partial(
    pl.kernel,
    out_type=[
        jax.ShapeDtypeStruct((_NT, _S), jnp.int32),
        jax.ShapeDtypeStruct((_NT, _S), jnp.int32),
    ],
    mesh=_mesh,
    compiler_params=_sc_params,
    scratch_types=[
        pltpu.VMEM((_EPT,), jnp.int32),
        pltpu.VMEM((_EPT,), jnp.int32),
        pltpu.VMEM((_EPT,), jnp.int32),
        pltpu.VMEM((_S,), jnp.int32),
        pltpu.VMEM((_S,), jnp.int32),
    ],
)
def _phase_a(src_h, rel_h, des_h, pk_h, cnt_h,
             src_v, rel_v, des_v, pk_v, cnt_v):
    wid = lax.axis_index("s") * 2 + lax.axis_index("c")
    base = wid * _EPT
    pltpu.sync_copy(src_h.at[pl.ds(base, _EPT)], src_v)
    pltpu.sync_copy(rel_h.at[pl.ds(base, _EPT)], rel_v)
    pltpu.sync_copy(des_h.at[pl.ds(base, _EPT)], des_v)

    zero = jnp.zeros((16,), jnp.int32)

    def init_body(i, _):
        for k in range(8):
            cnt_v[pl.ds((i * 8 + k) * 16, 16)] = zero
        return 0

    lax.fori_loop(0, _S // 128, init_body, 0)

    def body(i, _):
        for k in range(5):
            off = (i * 5 + k) * 16
            sl = pl.ds(off, 16)
            s = src_v[sl]
            r = rel_v[sl]
            d = des_v[sl]
            key = jnp.where(r < _NR, d, d + _NPAD)
            c16, last = plsc.scan_count(key)
            p = (s << 14) | r
            plsc.store_scatter(pk_v, [key], p, mask=last)
            plsc.addupdate_scatter(cnt_v, [key], c16, mask=last)
        return 0

    lax.fori_loop(0, _EPT // 80, body, 0)

    pltpu.sync_copy(pk_v, pk_h.at[wid])
    pltpu.sync_copy(cnt_v, cnt_h.at[wid])


@functools.partial(
    pl.kernel,
    out_type=[
        jax.ShapeDtypeStruct((2, _NPAD, _D), jnp.float32),
        jax.ShapeDtypeStruct((2, _NPAD), jnp.float32),
    ],
    mesh=_mesh,
    compiler_params=_sc_params,
    scratch_types=[
        pltpu.VMEM((_NT, _EB), jnp.int32),
        pltpu.VMEM((_NT, _EB), jnp.int32),
        pltpu.VMEM((_EB,), jnp.int32),
        pltpu.VMEM((_EB,), jnp.int32),
        pltpu.VMEM((_EB,), jnp.float32),
        pltpu.VMEM((_GCH, _D), jnp.float32),
        pltpu.VMEM((_GCH, _D), jnp.float32),
        pltpu.SemaphoreType.DMA,
        pltpu.SemaphoreType.DMA,
    ],
)
def _phase_b(pk_h, cnt_h, node_h, rela_h, u_h, c_h,
             pks, cnts, bsrc, brel, cf, rows_n, rows_r,
             sem1, sem2):
    wid = lax.axis_index("s") * 2 + lax.axis_index("c")
    col0 = wid * _EB
    half = wid // (_NPAD // _EB)
    row0 = col0 - half * _NPAD

    pltpu.sync_copy(pk_h.at[:, pl.ds(col0, _EB)], pks)
    pltpu.sync_copy(cnt_h.at[:, pl.ds(col0, _EB)], cnts)

    def mbody(i, _):
        off = i * 16
        z = jnp.zeros((16,), jnp.int32)
        bp = z
        cs = z
        for t in range(_NT):
            cv = cnts[t, pl.ds(off, 16)]
            bp = jnp.where(cv > 0, pks[t, pl.ds(off, 16)], bp)
            cs = cs + cv
        bsrc[pl.ds(off, 16)] = lax.shift_right_logical(bp, 14)
        brel[pl.ds(off, 16)] = bp & 16383
        cf[pl.ds(off, 16)] = cs.astype(jnp.float32)
        return 0

    lax.fori_loop(0, _EB // 16, mbody, 0)

    pltpu.sync_copy(cf, c_h.at[half, pl.ds(row0, _EB)])

    def gbody(g, _):
        goff = g * _GCH
        cp1 = pltpu.async_copy(
            node_h.at[bsrc.at[pl.ds(goff, _GCH)]], rows_n, sem1)
        cp2 = pltpu.async_copy(
            rela_h.at[brel.at[pl.ds(goff, _GCH)]], rows_r, sem2)
        cp1.wait()
        cp2.wait()

        def abody(j, _):
            for k in range(_D // 16):
                sl = pl.ds(k * 16, 16)
                rows_n[j, sl] = rows_n[j, sl] + rows_r[j, sl]
            return 0

        lax.fori_loop(0, _GCH, abody, 0)
        pltpu.sync_copy(rows_n, u_h.at[half, pl.ds(row0 + goff, _GCH)])
        return 0

    lax.fori_loop(0, _EB // _GCH, gbody, 0)


_DN = (((1,), (1,)), ((), ()))


def _tc1_body(node, r0, rela, wi, wr, base_o, hr_o):
    nb = node[...] + r0[...]
    base_o[...] = lax.dot_general(
        nb, wi[...], _DN, preferred_element_type=jnp.float32)
    hr_o[...] = lax.dot_general(
        rela[...], wr[...], _DN, preferred_element_type=jnp.float32)


def _tc2_body(base, u1, u2, c1, c2, wo, ws, hv_o):
    acc = base[...]
    acc += c1[...] * lax.dot_general(
        u1[0], wo[...], _DN, preferred_element_type=jnp.float32)
    acc += c2[...] * lax.dot_general(
        u2[0], ws[...], _DN, preferred_element_type=jnp.float32)
    hv_o[...] = acc


_NRE = 2 * _NR + 1

_tc1_call = pl.pallas_call(
    _tc1_body,
    grid=(_N // _BLK,),
    in_specs=[
        pl.BlockSpec((_BLK, _D), lambda i: (i, 0)),
        pl.BlockSpec((1, _D), lambda i: (0, 0)),
        pl.BlockSpec((_RBLK, _D), lambda i: (i, 0)),
        pl.BlockSpec((_D, _D), lambda i: (0, 0)),
        pl.BlockSpec((_D, _D), lambda i: (0, 0)),
    ],
    out_specs=[
        pl.BlockSpec((_BLK, _D), lambda i: (i, 0)),
        pl.BlockSpec((_RBLK, _D), lambda i: (i, 0)),
    ],
    out_shape=[
        jax.ShapeDtypeStruct((_N, _D), jnp.float32),
        jax.ShapeDtypeStruct((_NRE, _D), jnp.float32),
    ],
)

_tc2_call = pl.pallas_call(
    _tc2_body,
    grid=(_N // _BLK,),
    in_specs=[
        pl.BlockSpec((_BLK, _D), lambda i: (i, 0)),
        pl.BlockSpec((1, _BLK, _D), lambda i: (0, i, 0)),
        pl.BlockSpec((1, _BLK, _D), lambda i: (1, i, 0)),
        pl.BlockSpec((_BLK, 1), lambda i: (i, 0)),
        pl.BlockSpec((_BLK, 1), lambda i: (i, 0)),
        pl.BlockSpec((_D, _D), lambda i: (0, 0)),
        pl.BlockSpec((_D, _D), lambda i: (0, 0)),
    ],
    out_specs=pl.BlockSpec((_BLK, _D), lambda i: (i, 0)),
    out_shape=jax.ShapeDtypeStruct((_N, _D), jnp.float32),
)


def kernel(node_embed, rela_embed, edges, W_o, W_i, W_s, W_r):
    pk, cnt = _phase_a(edges[:, 0], edges[:, 1], edges[:, 2])
    u, c = _phase_b(pk, cnt, node_embed, rela_embed)

    r0 = rela_embed[_NRE - 1][None, :]
    base, hr = _tc1_call(node_embed, r0, rela_embed, W_i, W_r)
    c1 = c[0, :_N, None]
    c2 = c[1, :_N, None]
    hv = _tc2_call(base, u, u, c1, c2, W_o, W_s)
    return hv, hr
